# baseline (speedup 1.0000x reference)
import jax
import jax.numpy as jnp
from jax.experimental import pallas as pl
from jax.experimental.pallas import tpu as pltpu

_BLK = 256
_MU = 0.25
_BF = jnp.bfloat16
_F32 = jnp.float32


def _body(z_ref, q0_ref, q1_ref, q2_ref,
          dw0, db0, dw1, db1, dw2, db2,
          out_ref, loss_ref, xq_ref):
    i = pl.program_id(0)

    r = z_ref[...]
    xq = jnp.zeros_like(r)
    ss = jnp.zeros((), _F32)
    for q_ref in (q0_ref, q1_ref, q2_ref):
        q = q_ref[...]
        diff = q - r
        ss = ss + jnp.sum(diff * diff)
        q_st = r + diff
        xq = xq + q_st
        r = r - q_st

    xq_ref[...] = xq

    h = xq
    for j, (w, b) in enumerate(((dw0, db0), (dw1, db1), (dw2, db2))):
        h = jnp.dot(h.astype(_BF), w[...].astype(_BF),
                    preferred_element_type=_F32) + b[...]
        if j < 2:
            h = jnp.maximum(h, 0.0)
    out_ref[...] = h

    @pl.when(i == 0)
    def _init():
        loss_ref[...] = jnp.zeros_like(loss_ref)

    loss_ref[...] = loss_ref[...] + ss.reshape(1, 1)


def kernel(x, enc_Ws, enc_bs, dec_Ws, dec_bs, codebooks):
    B, in_dim = x.shape
    e_dim = enc_Ws[-1].shape[1]
    ncb = len(codebooks)
    grid = B // _BLK

    h = x
    for j, (W, b) in enumerate(zip(enc_Ws, enc_bs)):
        h = h @ W + b
        if j < len(enc_Ws) - 1:
            h = jax.nn.relu(h)
    z = h
    residual = z
    idx_list = []
    q_list = []
    for cb in codebooks:
        d = (jnp.sum(residual ** 2, axis=1, keepdims=True)
             + jnp.sum(cb ** 2, axis=1)[None, :]
             - 2.0 * (residual @ cb.T))
        idx = jnp.argmin(d, axis=1)
        q = jnp.take(cb, idx, axis=0)
        idx_list.append(idx)
        q_list.append(q)
        residual = residual - (residual + (q - residual))

    dbs = [b.reshape(1, -1) for b in dec_bs]

    def _full(a):
        return pl.BlockSpec(a.shape, lambda i: (0,) * a.ndim)

    dec_args = [a for pair in zip(dec_Ws, dbs) for a in pair]
    row_spec = pl.BlockSpec((_BLK, e_dim), lambda i: (i, 0))
    in_specs = [row_spec, row_spec, row_spec, row_spec] + [_full(a) for a in dec_args]

    out_shapes = (
        jax.ShapeDtypeStruct((B, in_dim), jnp.float32),
        jax.ShapeDtypeStruct((1, 1), jnp.float32),
        jax.ShapeDtypeStruct((B, e_dim), jnp.float32),
    )
    out_specs = (
        pl.BlockSpec((_BLK, in_dim), lambda i: (i, 0)),
        pl.BlockSpec((1, 1), lambda i: (0, 0)),
        pl.BlockSpec((_BLK, e_dim), lambda i: (i, 0)),
    )

    out, loss_sum, xq = pl.pallas_call(
        _body,
        grid=(grid,),
        in_specs=in_specs,
        out_specs=out_specs,
        out_shape=out_shapes,
        compiler_params=pltpu.CompilerParams(
            dimension_semantics=("arbitrary",)),
    )(z, *q_list, *dec_args)

    n_elem = B * e_dim
    rq_loss = loss_sum[0, 0] * ((1.0 + _MU) / (ncb * n_elem))
    indices = jnp.stack(idx_list, axis=-1).astype(jnp.int64)
    return out, rq_loss, indices, xq
